# baseline (speedup 1.0000x reference)
import functools

import jax
import jax.numpy as jnp
from jax.experimental import pallas as pl

B_ = 2
N_ = 20
C_ = 256
M_ = 250
K_ = 16
H_ = 256
MB = 256
P_ = N_ * M_
PP = N_ * MB
R_ = 512
RM = 512
CB = 512
NCH = PP // 128
BIG_D2 = 1.0e30
CONTRACT_0 = (((0,), (0,)), ((), ()))


def _nt_kernel(x_ref, nt_ref, sqc_ref):
    v = x_ref[0, 0]
    full = jnp.concatenate([v, jnp.zeros((C_, MB - M_), jnp.float32)], axis=1)
    nt_ref[0] = full
    sq = jnp.sum(full * full, axis=0, keepdims=True)
    col = jax.lax.broadcasted_iota(jnp.int32, (1, MB), 1)
    sqc_ref[0] = jnp.where(col < M_, sq, BIG_D2)


def _nth_min(v, n):
    for _ in range(n - 1):
        m = jnp.min(v, axis=1, keepdims=True)
        v = jnp.where(v <= m, jnp.inf, v)
    return jnp.min(v, axis=1, keepdims=True)


def _mask_kernel(ntb_ref, nt_ref, sqc_ref, mask_ref):
    ntb = ntb_ref[0]
    nt = nt_ref[0]
    g = jax.lax.dot_general(ntb, nt, CONTRACT_0,
                            preferred_element_type=jnp.float32)
    d2 = sqc_ref[0] - 2.0 * g

    inf = jnp.full((RM, 128), jnp.inf, jnp.float32)
    m1, m2, m3, m4 = inf, inf, inf, inf
    for k in range(NCH):
        c = d2[:, k * 128:(k + 1) * 128]
        t = jnp.maximum(m1, c)
        m1 = jnp.minimum(m1, c)
        u = jnp.maximum(m2, t)
        m2 = jnp.minimum(m2, t)
        w = jnp.maximum(m3, u)
        m3 = jnp.minimum(m3, u)
        m4 = jnp.minimum(m4, w)
    cand = jnp.concatenate([m1, m2, m3, m4], axis=1)
    t_hat = _nth_min(cand, K_ + 1)

    row = jax.lax.broadcasted_iota(jnp.int32, (RM, 1), 0)
    row_ok = (jax.lax.rem(row, MB) < M_).astype(jnp.float32)
    maskf = jnp.where(d2 <= t_hat, row_ok, 0.0)
    cnt = jnp.sum(maskf, axis=1, keepdims=True)
    badness = jnp.sum(jnp.abs(cnt - float(K_ + 1)) * row_ok)
    mask_ref[0] = maskf.astype(jnp.bfloat16)

    @pl.when(badness != 0.0)
    def _fallback():
        t = _nth_min(d2, K_ + 1)
        mask_ref[0] = jnp.where(d2 <= t, row_ok, 0.0).astype(jnp.bfloat16)


def _proj_t_kernel(ntb_ref, w_ref, as_ref, ad_ref, h_ref, asrc_ref, adst_ref):
    h = jax.lax.dot_general(ntb_ref[0], w_ref[...], CONTRACT_0,
                            preferred_element_type=jnp.float32)
    h_ref[0] = h.astype(jnp.bfloat16)
    asrc_ref[0] = jnp.sum(h * as_ref[...], axis=1, keepdims=True)
    adst_ref[0] = jnp.transpose(
        jnp.sum(h * ad_ref[...], axis=1, keepdims=True))


def _proj_kernel(x_ref, w_ref, as_ref, ad_ref, h_ref, asrc_ref, adst_ref):
    h = jnp.dot(x_ref[0], w_ref[...], preferred_element_type=jnp.float32)
    h_ref[0] = h.astype(jnp.bfloat16)
    asrc_ref[0] = jnp.sum(h * as_ref[...], axis=1, keepdims=True)
    adst_ref[0] = jnp.transpose(
        jnp.sum(h * ad_ref[...], axis=1, keepdims=True))


def _gat_kernel(h_ref, asrc_ref, adst_ref, b_ref, mask_ref, out_ref, *,
                apply_elu):
    h = h_ref[0]
    asrc = asrc_ref[0]
    adst = adst_ref[0]
    e = asrc + adst
    e = jnp.maximum(e, 0.2 * e)
    ex = jnp.exp(e).astype(jnp.bfloat16) * mask_ref[0]
    ones = jnp.ones((PP, 128), jnp.bfloat16)
    den = jax.lax.dot_general(ex, ones, CONTRACT_0,
                              preferred_element_type=jnp.float32)
    num = jax.lax.dot_general(ex, h, CONTRACT_0,
                              preferred_element_type=jnp.float32)
    rec = 1.0 / (den[:, 0:1] + 1e-16)
    out = num * rec + b_ref[...]
    if apply_elu:
        out = jnp.where(out > 0.0, out, jnp.exp(jnp.minimum(out, 0.0)) - 1.0)
    out_ref[0] = out


def _compact_kernel(i_ref, o_ref):
    o_ref[0, 0] = i_ref[0][:M_]


def _to_nt(x, *, interpret=False):
    return pl.pallas_call(
        _nt_kernel,
        grid=(B_, N_),
        in_specs=[pl.BlockSpec((1, 1, C_, M_), lambda b, n: (b, n, 0, 0))],
        out_specs=[
            pl.BlockSpec((1, C_, MB), lambda b, n: (b, 0, n)),
            pl.BlockSpec((1, 1, MB), lambda b, n: (b, 0, n)),
        ],
        out_shape=[
            jax.ShapeDtypeStruct((B_, C_, PP), jnp.float32),
            jax.ShapeDtypeStruct((B_, 1, PP), jnp.float32),
        ],
        interpret=interpret,
    )(x)


def _build_mask(nt, sqc, *, interpret=False):
    return pl.pallas_call(
        _mask_kernel,
        grid=(B_, PP // RM),
        in_specs=[
            pl.BlockSpec((1, C_, RM), lambda b, i: (b, 0, i)),
            pl.BlockSpec((1, C_, PP), lambda b, i: (b, 0, 0)),
            pl.BlockSpec((1, 1, PP), lambda b, i: (b, 0, 0)),
        ],
        out_specs=pl.BlockSpec((1, RM, PP), lambda b, i: (b, i, 0)),
        out_shape=jax.ShapeDtypeStruct((B_, PP, PP), jnp.bfloat16),
        interpret=interpret,
    )(nt, nt, sqc)


def _project(xn, w, a_s, a_d, *, transposed, interpret=False):
    if transposed:
        body, spec = _proj_t_kernel, pl.BlockSpec((1, C_, R_),
                                                  lambda b, i: (b, 0, i))
    else:
        body, spec = _proj_kernel, pl.BlockSpec((1, R_, H_),
                                                lambda b, i: (b, i, 0))
    return pl.pallas_call(
        body,
        grid=(B_, PP // R_),
        in_specs=[
            spec,
            pl.BlockSpec((C_, H_), lambda b, i: (0, 0)),
            pl.BlockSpec((1, H_), lambda b, i: (0, 0)),
            pl.BlockSpec((1, H_), lambda b, i: (0, 0)),
        ],
        out_specs=[
            pl.BlockSpec((1, R_, H_), lambda b, i: (b, i, 0)),
            pl.BlockSpec((1, R_, 1), lambda b, i: (b, i, 0)),
            pl.BlockSpec((1, 1, R_), lambda b, i: (b, 0, i)),
        ],
        out_shape=[
            jax.ShapeDtypeStruct((B_, PP, H_), jnp.bfloat16),
            jax.ShapeDtypeStruct((B_, PP, 1), jnp.float32),
            jax.ShapeDtypeStruct((B_, 1, PP), jnp.float32),
        ],
        interpret=interpret,
    )(xn, w, a_s, a_d)


def _gat_layer(h, asrc, adst, b, mask, *, apply_elu, interpret=False):
    return pl.pallas_call(
        functools.partial(_gat_kernel, apply_elu=apply_elu),
        grid=(B_, PP // CB),
        in_specs=[
            pl.BlockSpec((1, PP, H_), lambda b_, j: (b_, 0, 0)),
            pl.BlockSpec((1, PP, 1), lambda b_, j: (b_, 0, 0)),
            pl.BlockSpec((1, 1, CB), lambda b_, j: (b_, 0, j)),
            pl.BlockSpec((1, H_), lambda b_, j: (0, 0)),
            pl.BlockSpec((1, PP, CB), lambda b_, j: (b_, 0, j)),
        ],
        out_specs=pl.BlockSpec((1, CB, H_), lambda b_, j: (b_, j, 0)),
        out_shape=jax.ShapeDtypeStruct((B_, PP, H_), jnp.float32),
        interpret=interpret,
    )(h, asrc, adst, b, mask)


def _compact(out2, *, interpret=False):
    return pl.pallas_call(
        _compact_kernel,
        grid=(B_, N_),
        in_specs=[pl.BlockSpec((1, MB, H_), lambda b, n: (b, n, 0))],
        out_specs=pl.BlockSpec((1, 1, M_, H_), lambda b, n: (b, n, 0, 0)),
        out_shape=jax.ShapeDtypeStruct((B_, N_, M_, H_), jnp.float32),
        interpret=interpret,
    )(out2)


def _run(x, W1, att_src1, att_dst1, b1, W2, att_src2, att_dst2, b2,
         interpret=False):
    nt, sqc = _to_nt(x, interpret=interpret)

    mask = _build_mask(nt, sqc, interpret=interpret)

    h1, asrc1, adst1 = _project(nt, W1, att_src1.reshape(1, H_),
                                att_dst1.reshape(1, H_), transposed=True,
                                interpret=interpret)
    out1 = _gat_layer(h1, asrc1, adst1, b1.reshape(1, H_), mask,
                      apply_elu=True, interpret=interpret)
    h2, asrc2, adst2 = _project(out1, W2, att_src2.reshape(1, H_),
                                att_dst2.reshape(1, H_), transposed=False,
                                interpret=interpret)
    out2 = _gat_layer(h2, asrc2, adst2, b2.reshape(1, H_), mask,
                      apply_elu=False, interpret=interpret)
    return _compact(out2, interpret=interpret)


def kernel(x, W1, att_src1, att_dst1, b1, W2, att_src2, att_dst2, b2):
    return _run(x, W1, att_src1, att_dst1, b1, W2, att_src2, att_dst2, b2)


# speedup vs baseline: 1.0871x; 1.0871x over previous
import functools

import jax
import jax.numpy as jnp
from jax.experimental import pallas as pl

B_ = 2
N_ = 20
C_ = 256
M_ = 250
K_ = 16
H_ = 256
MB = 256
P_ = N_ * M_
PP = N_ * MB
R_ = 512
RM = 512
CB = 512
NCH = PP // 128
BIG_D2 = 1.0e30
CONTRACT_0 = (((0,), (0,)), ((), ()))


def _nt_kernel(x_ref, nt_ref, sqc_ref):
    v = x_ref[0, 0]
    full = jnp.concatenate([v, jnp.zeros((C_, MB - M_), jnp.float32)], axis=1)
    nt_ref[0] = full
    sq = jnp.sum(full * full, axis=0, keepdims=True)
    col = jax.lax.broadcasted_iota(jnp.int32, (1, MB), 1)
    sqc_ref[0] = jnp.where(col < M_, sq, BIG_D2)


def _nth_min(v, n):
    for _ in range(n - 1):
        m = jnp.min(v, axis=1, keepdims=True)
        v = jnp.where(v <= m, jnp.inf, v)
    return jnp.min(v, axis=1, keepdims=True)


def _mask_kernel(ntb_ref, nt_ref, sqc_ref, mask_ref):
    ntb = ntb_ref[0]
    nt = nt_ref[0]
    g = jax.lax.dot_general(ntb, nt, CONTRACT_0,
                            preferred_element_type=jnp.float32)
    d2 = sqc_ref[0] - 2.0 * g

    inf = jnp.full((RM, 128), jnp.inf, jnp.float32)
    m1, m2, m3, m4 = inf, inf, inf, inf
    for k in range(NCH):
        c = d2[:, k * 128:(k + 1) * 128]
        t = jnp.maximum(m1, c)
        m1 = jnp.minimum(m1, c)
        u = jnp.maximum(m2, t)
        m2 = jnp.minimum(m2, t)
        w = jnp.maximum(m3, u)
        m3 = jnp.minimum(m3, u)
        m4 = jnp.minimum(m4, w)
    cand = jnp.concatenate([m1, m2, m3, m4], axis=1)
    t_hat = _nth_min(cand, K_ + 1)

    row = jax.lax.broadcasted_iota(jnp.int32, (RM, 1), 0)
    row_ok = (jax.lax.rem(row, MB) < M_).astype(jnp.float32)
    maskf = jnp.where(d2 <= t_hat, row_ok, 0.0)
    cnt = jnp.sum(maskf, axis=1, keepdims=True)
    badness = jnp.sum(jnp.abs(cnt - float(K_ + 1)) * row_ok)
    mask_ref[0] = maskf.astype(jnp.bfloat16)

    @pl.when(badness != 0.0)
    def _fallback():
        t = _nth_min(d2, K_ + 1)
        mask_ref[0] = jnp.where(d2 <= t, row_ok, 0.0).astype(jnp.bfloat16)


def _proj_t_kernel(ntb_ref, w_ref, as_ref, ad_ref, h_ref, asrc_ref, adst_ref):
    h = jax.lax.dot_general(ntb_ref[0], w_ref[...], CONTRACT_0,
                            preferred_element_type=jnp.float32)
    h_ref[0] = h.astype(jnp.bfloat16)
    asrc_ref[0] = jnp.sum(h * as_ref[...], axis=1, keepdims=True)
    adst_ref[0] = jnp.transpose(
        jnp.sum(h * ad_ref[...], axis=1, keepdims=True))


def _proj_kernel(x_ref, w_ref, as_ref, ad_ref, h_ref, asrc_ref, adst_ref):
    h = jnp.dot(x_ref[0], w_ref[...], preferred_element_type=jnp.float32)
    h_ref[0] = h.astype(jnp.bfloat16)
    asrc_ref[0] = jnp.sum(h * as_ref[...], axis=1, keepdims=True)
    adst_ref[0] = jnp.transpose(
        jnp.sum(h * ad_ref[...], axis=1, keepdims=True))


def _gat_kernel(h_ref, asrc_ref, adst_ref, b_ref, mask_ref, out_ref, *,
                apply_elu):
    h = h_ref[0]
    asrc = asrc_ref[0]
    adst = adst_ref[0]
    e = asrc + adst
    e = jnp.maximum(e, 0.2 * e)
    ex = jnp.exp(e) * mask_ref[0].astype(jnp.float32)
    denom = jnp.sum(ex, axis=0, keepdims=True)
    num = jax.lax.dot_general(ex.astype(jnp.bfloat16), h, CONTRACT_0,
                              preferred_element_type=jnp.float32)
    rec = jnp.transpose(1.0 / (denom + 1e-16))
    out = num * rec + b_ref[...]
    if apply_elu:
        out = jnp.where(out > 0.0, out, jnp.exp(jnp.minimum(out, 0.0)) - 1.0)
    out_ref[0] = out


def _compact_kernel(i_ref, o_ref):
    o_ref[0, 0] = i_ref[0][:M_]


def _to_nt(x, *, interpret=False):
    return pl.pallas_call(
        _nt_kernel,
        grid=(B_, N_),
        in_specs=[pl.BlockSpec((1, 1, C_, M_), lambda b, n: (b, n, 0, 0))],
        out_specs=[
            pl.BlockSpec((1, C_, MB), lambda b, n: (b, 0, n)),
            pl.BlockSpec((1, 1, MB), lambda b, n: (b, 0, n)),
        ],
        out_shape=[
            jax.ShapeDtypeStruct((B_, C_, PP), jnp.float32),
            jax.ShapeDtypeStruct((B_, 1, PP), jnp.float32),
        ],
        interpret=interpret,
    )(x)


def _build_mask(nt, sqc, *, interpret=False):
    return pl.pallas_call(
        _mask_kernel,
        grid=(B_, PP // RM),
        in_specs=[
            pl.BlockSpec((1, C_, RM), lambda b, i: (b, 0, i)),
            pl.BlockSpec((1, C_, PP), lambda b, i: (b, 0, 0)),
            pl.BlockSpec((1, 1, PP), lambda b, i: (b, 0, 0)),
        ],
        out_specs=pl.BlockSpec((1, RM, PP), lambda b, i: (b, i, 0)),
        out_shape=jax.ShapeDtypeStruct((B_, PP, PP), jnp.bfloat16),
        interpret=interpret,
    )(nt, nt, sqc)


def _project(xn, w, a_s, a_d, *, transposed, interpret=False):
    if transposed:
        body, spec = _proj_t_kernel, pl.BlockSpec((1, C_, R_),
                                                  lambda b, i: (b, 0, i))
    else:
        body, spec = _proj_kernel, pl.BlockSpec((1, R_, H_),
                                                lambda b, i: (b, i, 0))
    return pl.pallas_call(
        body,
        grid=(B_, PP // R_),
        in_specs=[
            spec,
            pl.BlockSpec((C_, H_), lambda b, i: (0, 0)),
            pl.BlockSpec((1, H_), lambda b, i: (0, 0)),
            pl.BlockSpec((1, H_), lambda b, i: (0, 0)),
        ],
        out_specs=[
            pl.BlockSpec((1, R_, H_), lambda b, i: (b, i, 0)),
            pl.BlockSpec((1, R_, 1), lambda b, i: (b, i, 0)),
            pl.BlockSpec((1, 1, R_), lambda b, i: (b, 0, i)),
        ],
        out_shape=[
            jax.ShapeDtypeStruct((B_, PP, H_), jnp.bfloat16),
            jax.ShapeDtypeStruct((B_, PP, 1), jnp.float32),
            jax.ShapeDtypeStruct((B_, 1, PP), jnp.float32),
        ],
        interpret=interpret,
    )(xn, w, a_s, a_d)


def _gat_layer(h, asrc, adst, b, mask, *, apply_elu, interpret=False):
    return pl.pallas_call(
        functools.partial(_gat_kernel, apply_elu=apply_elu),
        grid=(B_, PP // CB),
        in_specs=[
            pl.BlockSpec((1, PP, H_), lambda b_, j: (b_, 0, 0)),
            pl.BlockSpec((1, PP, 1), lambda b_, j: (b_, 0, 0)),
            pl.BlockSpec((1, 1, CB), lambda b_, j: (b_, 0, j)),
            pl.BlockSpec((1, H_), lambda b_, j: (0, 0)),
            pl.BlockSpec((1, PP, CB), lambda b_, j: (b_, 0, j)),
        ],
        out_specs=pl.BlockSpec((1, CB, H_), lambda b_, j: (b_, j, 0)),
        out_shape=jax.ShapeDtypeStruct((B_, PP, H_), jnp.float32),
        interpret=interpret,
    )(h, asrc, adst, b, mask)


def _compact(out2, *, interpret=False):
    return pl.pallas_call(
        _compact_kernel,
        grid=(B_, N_),
        in_specs=[pl.BlockSpec((1, MB, H_), lambda b, n: (b, n, 0))],
        out_specs=pl.BlockSpec((1, 1, M_, H_), lambda b, n: (b, n, 0, 0)),
        out_shape=jax.ShapeDtypeStruct((B_, N_, M_, H_), jnp.float32),
        interpret=interpret,
    )(out2)


def _run(x, W1, att_src1, att_dst1, b1, W2, att_src2, att_dst2, b2,
         interpret=False):
    nt, sqc = _to_nt(x, interpret=interpret)

    mask = _build_mask(nt, sqc, interpret=interpret)

    h1, asrc1, adst1 = _project(nt, W1, att_src1.reshape(1, H_),
                                att_dst1.reshape(1, H_), transposed=True,
                                interpret=interpret)
    out1 = _gat_layer(h1, asrc1, adst1, b1.reshape(1, H_), mask,
                      apply_elu=True, interpret=interpret)
    h2, asrc2, adst2 = _project(out1, W2, att_src2.reshape(1, H_),
                                att_dst2.reshape(1, H_), transposed=False,
                                interpret=interpret)
    out2 = _gat_layer(h2, asrc2, adst2, b2.reshape(1, H_), mask,
                      apply_elu=False, interpret=interpret)
    return _compact(out2, interpret=interpret)


def kernel(x, W1, att_src1, att_dst1, b1, W2, att_src2, att_dst2, b2):
    return _run(x, W1, att_src1, att_dst1, b1, W2, att_src2, att_dst2, b2)


# speedup vs baseline: 1.1339x; 1.0430x over previous
import functools

import jax
import jax.numpy as jnp
from jax.experimental import pallas as pl

B_ = 2
N_ = 20
C_ = 256
M_ = 250
K_ = 16
H_ = 256
MB = 256
P_ = N_ * M_
PP = N_ * MB
R_ = 512
RM = 512
CB = 512
NCH = PP // 128
BIG_D2 = 1.0e30
CONTRACT_0 = (((0,), (0,)), ((), ()))


def _nt_kernel(x_ref, nt_ref, sqc_ref):
    v = x_ref[0, 0]
    full = jnp.concatenate([v, jnp.zeros((C_, MB - M_), jnp.float32)], axis=1)
    nt_ref[0] = full
    sq = jnp.sum(full * full, axis=0, keepdims=True)
    col = jax.lax.broadcasted_iota(jnp.int32, (1, MB), 1)
    sqc_ref[0] = jnp.where(col < M_, sq, BIG_D2)


def _nth_min(v, n):
    for _ in range(n - 1):
        m = jnp.min(v, axis=1, keepdims=True)
        v = jnp.where(v <= m, jnp.inf, v)
    return jnp.min(v, axis=1, keepdims=True)


def _mask_kernel(ntb_ref, nt_ref, sqc_ref, mask_ref):
    ntb = ntb_ref[0]
    nt = nt_ref[0]
    g = jax.lax.dot_general(ntb, nt, CONTRACT_0,
                            preferred_element_type=jnp.float32)
    d2 = sqc_ref[0] - 2.0 * g

    inf = jnp.full((RM, 128), jnp.inf, jnp.float32)
    m1, m2, m3, m4 = inf, inf, inf, inf
    for k in range(NCH):
        c = d2[:, k * 128:(k + 1) * 128]
        t = jnp.maximum(m1, c)
        m1 = jnp.minimum(m1, c)
        u = jnp.maximum(m2, t)
        m2 = jnp.minimum(m2, t)
        w = jnp.maximum(m3, u)
        m3 = jnp.minimum(m3, u)
        m4 = jnp.minimum(m4, w)
    cand = jnp.concatenate([m1, m2, m3, m4], axis=1)
    t_hat = _nth_min(cand, K_ + 1)

    row = jax.lax.broadcasted_iota(jnp.int32, (RM, 1), 0)
    row_ok = (jax.lax.rem(row, MB) < M_).astype(jnp.float32)
    maskf = jnp.where(d2 <= t_hat, row_ok, 0.0)
    cnt = jnp.sum(maskf, axis=1, keepdims=True)
    badness = jnp.sum(jnp.abs(cnt - float(K_ + 1)) * row_ok)
    mask_ref[0] = maskf.astype(jnp.bfloat16)

    @pl.when(badness != 0.0)
    def _fallback():
        t = _nth_min(d2, K_ + 1)
        mask_ref[0] = jnp.where(d2 <= t, row_ok, 0.0).astype(jnp.bfloat16)


def _proj_t_kernel(ntb_ref, w_ref, as_ref, ad_ref, h_ref, asrc_ref, adst_ref):
    h = jax.lax.dot_general(ntb_ref[0], w_ref[...], CONTRACT_0,
                            preferred_element_type=jnp.float32)
    h_ref[0] = h.astype(jnp.bfloat16)
    asrc_ref[0] = jnp.sum(h * as_ref[...], axis=1, keepdims=True)
    adst_ref[0] = jnp.transpose(
        jnp.sum(h * ad_ref[...], axis=1, keepdims=True))


def _proj_kernel(x_ref, w_ref, as_ref, ad_ref, h_ref, asrc_ref, adst_ref):
    h = jnp.dot(x_ref[0], w_ref[...], preferred_element_type=jnp.float32)
    h_ref[0] = h.astype(jnp.bfloat16)
    asrc_ref[0] = jnp.sum(h * as_ref[...], axis=1, keepdims=True)
    adst_ref[0] = jnp.transpose(
        jnp.sum(h * ad_ref[...], axis=1, keepdims=True))


def _gat_kernel(h_ref, asrc_ref, adst_ref, b_ref, mask_ref, out_ref, *,
                apply_elu):
    h = h_ref[0]
    asrc = asrc_ref[0]
    adst = adst_ref[0]
    e = asrc + adst
    e = jnp.maximum(e, 0.2 * e)
    ex = jnp.exp(e) * mask_ref[0].astype(jnp.float32)
    denom = jnp.sum(ex, axis=0, keepdims=True)
    num = jax.lax.dot_general(ex.astype(jnp.bfloat16), h, CONTRACT_0,
                              preferred_element_type=jnp.float32)
    rec = jnp.transpose(1.0 / (denom + 1e-16))
    out = num * rec + b_ref[...]
    if apply_elu:
        out = jnp.where(out > 0.0, out, jnp.exp(jnp.minimum(out, 0.0)) - 1.0)
    out_ref[0] = out


def _compact_kernel(i_ref, o_ref):
    o_ref[0, 0] = i_ref[0][:M_]


def _to_nt(x, *, interpret=False):
    return pl.pallas_call(
        _nt_kernel,
        grid=(B_, N_),
        in_specs=[pl.BlockSpec((1, 1, C_, M_), lambda b, n: (b, n, 0, 0))],
        out_specs=[
            pl.BlockSpec((1, C_, MB), lambda b, n: (b, 0, n)),
            pl.BlockSpec((1, 1, MB), lambda b, n: (b, 0, n)),
        ],
        out_shape=[
            jax.ShapeDtypeStruct((B_, C_, PP), jnp.float32),
            jax.ShapeDtypeStruct((B_, 1, PP), jnp.float32),
        ],
        interpret=interpret,
    )(x)


def _build_mask(nt, sqc, *, interpret=False):
    return pl.pallas_call(
        _mask_kernel,
        grid=(B_, PP // RM),
        in_specs=[
            pl.BlockSpec((1, C_, RM), lambda b, i: (b, 0, i)),
            pl.BlockSpec((1, C_, PP), lambda b, i: (b, 0, 0)),
            pl.BlockSpec((1, 1, PP), lambda b, i: (b, 0, 0)),
        ],
        out_specs=pl.BlockSpec((1, RM, PP), lambda b, i: (b, i, 0)),
        out_shape=jax.ShapeDtypeStruct((B_, PP, PP), jnp.bfloat16),
        interpret=interpret,
    )(nt, nt, sqc)


def _project(xn, w, a_s, a_d, *, transposed, interpret=False):
    if transposed:
        body, spec = _proj_t_kernel, pl.BlockSpec((1, C_, R_),
                                                  lambda b, i: (b, 0, i))
    else:
        body, spec = _proj_kernel, pl.BlockSpec((1, R_, H_),
                                                lambda b, i: (b, i, 0))
    return pl.pallas_call(
        body,
        grid=(B_, PP // R_),
        in_specs=[
            spec,
            pl.BlockSpec((C_, H_), lambda b, i: (0, 0)),
            pl.BlockSpec((1, H_), lambda b, i: (0, 0)),
            pl.BlockSpec((1, H_), lambda b, i: (0, 0)),
        ],
        out_specs=[
            pl.BlockSpec((1, R_, H_), lambda b, i: (b, i, 0)),
            pl.BlockSpec((1, R_, 1), lambda b, i: (b, i, 0)),
            pl.BlockSpec((1, 1, R_), lambda b, i: (b, 0, i)),
        ],
        out_shape=[
            jax.ShapeDtypeStruct((B_, PP, H_), jnp.bfloat16),
            jax.ShapeDtypeStruct((B_, PP, 1), jnp.float32),
            jax.ShapeDtypeStruct((B_, 1, PP), jnp.float32),
        ],
        interpret=interpret,
    )(xn, w, a_s, a_d)


def _gat_layer(h, asrc, adst, b, mask, *, apply_elu, interpret=False):
    return pl.pallas_call(
        functools.partial(_gat_kernel, apply_elu=apply_elu),
        grid=(B_, PP // CB),
        in_specs=[
            pl.BlockSpec((1, PP, H_), lambda b_, j: (b_, 0, 0)),
            pl.BlockSpec((1, PP, 1), lambda b_, j: (b_, 0, 0)),
            pl.BlockSpec((1, 1, CB), lambda b_, j: (b_, 0, j)),
            pl.BlockSpec((1, H_), lambda b_, j: (0, 0)),
            pl.BlockSpec((1, PP, CB), lambda b_, j: (b_, 0, j)),
        ],
        out_specs=pl.BlockSpec((1, CB, H_), lambda b_, j: (b_, j, 0)),
        out_shape=jax.ShapeDtypeStruct((B_, PP, H_), jnp.float32),
        interpret=interpret,
    )(h, asrc, adst, b, mask)


def _compact(out2, *, interpret=False):
    return pl.pallas_call(
        _compact_kernel,
        grid=(B_, N_),
        in_specs=[pl.BlockSpec((1, MB, H_), lambda b, n: (b, n, 0))],
        out_specs=pl.BlockSpec((1, 1, M_, H_), lambda b, n: (b, n, 0, 0)),
        out_shape=jax.ShapeDtypeStruct((B_, N_, M_, H_), jnp.float32),
        interpret=interpret,
    )(out2)


def _run(x, W1, att_src1, att_dst1, b1, W2, att_src2, att_dst2, b2,
         interpret=False):
    nt, sqc = _to_nt(x, interpret=interpret)

    mask = _build_mask(nt, sqc, interpret=interpret)

    h1, asrc1, adst1 = _project(nt, W1, att_src1.reshape(1, H_),
                                att_dst1.reshape(1, H_), transposed=True,
                                interpret=interpret)
    out1 = _gat_layer(h1, asrc1, adst1, b1.reshape(1, H_), mask,
                      apply_elu=True, interpret=interpret)
    h2, asrc2, adst2 = _project(out1, W2, att_src2.reshape(1, H_),
                                att_dst2.reshape(1, H_), transposed=False,
                                interpret=interpret)
    out2 = _gat_layer(h2, asrc2, adst2, b2.reshape(1, H_), mask,
                      apply_elu=False, interpret=interpret)
    return out2.reshape(B_, N_, MB, H_)[:, :, :M_, :]


def kernel(x, W1, att_src1, att_dst1, b1, W2, att_src2, att_dst2, b2):
    return _run(x, W1, att_src1, att_dst1, b1, W2, att_src2, att_dst2, b2)
